# baseline (speedup 1.0000x reference)
import functools

import jax
import jax.numpy as jnp
from jax import lax
from jax.experimental import pallas as pl
from jax.experimental.pallas import tpu as pltpu
from jax.experimental.pallas import tpu_sc as plsc

N_MID = 1000000
DIM = 64
SEQ = 50
BATCH = 4096

NC = 2
NS = 16
NW = NC * NS

ROWS = BATCH * SEQ
BPW = BATCH // NW
BPC = 4
C = BPC * SEQ
CH = BPW // BPC
NLANE = DIM // 16

TBLOCK = 1024
NBLOCKS = 500
SROWS = NBLOCKS * TBLOCK
RSHIFT = (NBLOCKS - 1) * TBLOCK
EDGE_BLOCK = N_MID // TBLOCK

G0, G1 = 128, C - 128

GROUPS = ((0, 0, 16), (16, 0, 16), (32, 0, 16), (34, 14, 16))
IDX_GROUPS = tuple(g * 16 for g in range(12)) + (C - 16,)


def _stage_body(in1_ref, in2_ref, out_ref):
    out_ref[:, 0:DIM] = in1_ref[...].T
    out_ref[:, DIM:2 * DIM] = in2_ref[...].T


_stage = pl.pallas_call(
    _stage_body,
    grid=(NBLOCKS,),
    in_specs=[
        pl.BlockSpec((DIM, TBLOCK), lambda k: (0, k)),
        pl.BlockSpec(
            (DIM, TBLOCK),
            lambda k: (0, jnp.minimum(k + NBLOCKS - 1, EDGE_BLOCK)),
        ),
    ],
    out_specs=pl.BlockSpec((TBLOCK, 2 * DIM), lambda k: (k, 0)),
    out_shape=jax.ShapeDtypeStruct((SROWS, 2 * DIM), jnp.float32),
)


@functools.partial(
    pl.kernel,
    out_type=jax.ShapeDtypeStruct((ROWS, DIM), jnp.float32),
    mesh=plsc.VectorSubcoreMesh(core_axis_name="c", subcore_axis_name="s"),
    scratch_types=[
        pltpu.VMEM((C,), jnp.int32),
        pltpu.VMEM((C,), jnp.int32),
        pltpu.VMEM((C,), jnp.int32),
        pltpu.VMEM((C,), jnp.int32),
        pltpu.VMEM((C,), jnp.int32),
        pltpu.VMEM((C,), jnp.int32),
        pltpu.VMEM((2, C, 2 * DIM), jnp.float32),
        pltpu.VMEM((2, C, DIM), jnp.float32),
        pltpu.VMEM((SEQ, DIM), jnp.float32),
        pltpu.SemaphoreType.DMA,
        pltpu.SemaphoreType.DMA,
        pltpu.SemaphoreType.DMA,
        pltpu.SemaphoreType.DMA,
    ],
)
def _sc_lookup(item_hbm, pos_hbm, table_hbm, out_hbm,
               idx0, idx1, gidx0, gidx1, off0, off1, gbuf, obuf, pos_v,
               gsem0, gsem1, ssem0, ssem1):
    cid = lax.axis_index("c")
    sid = lax.axis_index("s")
    wid = sid * NC + cid
    idxs = (idx0, idx1)
    gidxs = (gidx0, gidx1)
    offs = (off0, off1)
    gsem = (gsem0, gsem1)
    ssem = (ssem0, ssem1)

    pltpu.sync_copy(pos_hbm, pos_v)

    def fire_chunk(t, par):
        base = wid * BPW * SEQ + t * C
        pltpu.sync_copy(item_hbm.at[pl.ds(base, C)], idxs[par])
        for off in IDX_GROUPS:
            v = idxs[par][pl.ds(off, 16)]
            ge = v >= SROWS
            gidxs[par][pl.ds(off, 16)] = jnp.where(ge, v - RSHIFT, v)
            offs[par][pl.ds(off, 16)] = jnp.where(ge, DIM, 0)
        pltpu.async_copy(
            table_hbm.at[gidxs[par].at[pl.ds(0, G0)]],
            gbuf.at[par, pl.ds(0, G0)],
            gsem[par],
        )
        pltpu.async_copy(
            table_hbm.at[gidxs[par].at[pl.ds(G0, G1)]],
            gbuf.at[par, pl.ds(G0, G1)],
            gsem[par],
        )

    def drain_gather(par):
        pltpu.make_async_copy(
            table_hbm.at[pl.ds(0, C)], gbuf.at[par], gsem[par]
        ).wait()

    def drain_store(par):
        pltpu.make_async_copy(
            obuf.at[par], out_hbm.at[pl.ds(0, C)], ssem[par]
        ).wait()

    fire_chunk(0, 0)

    def step(t, par):
        other = 1 - par

        @pl.when(t >= 1)
        def _():
            drain_store(other)

        @pl.when(t + 1 < CH)
        def _():
            fire_chunk(t + 1, other)

        drain_gather(par)

        for b in range(BPC):
            boff = b * SEQ
            for off_g, lane_lo, lane_hi in GROUPS:
                hvec = offs[par][pl.ds(boff + off_g, 16)]
                for lane in range(lane_lo, lane_hi):
                    s = off_g + lane
                    r = boff + s
                    half = hvec[lane]
                    for d in range(NLANE):
                        sl = pl.ds(d * 16, 16)
                        src = pl.multiple_of(half + d * 16, 16)
                        obuf[par, r, sl] = (
                            gbuf[par, r, pl.ds(src, 16)]
                            + pos_v[s, sl]
                        )

        base = wid * BPW * SEQ + t * C
        pltpu.async_copy(obuf.at[par], out_hbm.at[pl.ds(base, C)], ssem[par])

    def pair(tt, carry):
        step(tt * 2, 0)
        step(tt * 2 + 1, 1)
        return carry

    lax.fori_loop(0, CH // 2, pair, 0)
    drain_store((CH - 1) % 2)


def kernel(item, nbr_mask, i_ids, item_input_lookup, position_embedding):
    idx_flat = item.reshape(-1)
    tt = item_input_lookup.T
    staged = _stage(tt, tt)
    pos = position_embedding.reshape(SEQ, DIM)
    out = _sc_lookup(idx_flat, pos, staged)
    return out.reshape(BATCH, SEQ, DIM)


# speedup vs baseline: 1.6618x; 1.6618x over previous
import functools

import jax
import jax.numpy as jnp
from jax import lax
from jax.experimental import pallas as pl
from jax.experimental.pallas import tpu as pltpu
from jax.experimental.pallas import tpu_sc as plsc

N_MID = 1000000
DIM = 64
SEQ = 50
BATCH = 4096

NC = 2
NS = 16
NW = NC * NS

BPW = BATCH // NW
BPC = 4
C = BPC * SEQ
CH = BPW // BPC
NLANE = DIM // 16

GROUPS = ((0, 0, 16), (16, 0, 16), (32, 0, 16), (34, 14, 16))


@functools.partial(
    pl.kernel,
    out_type=jax.ShapeDtypeStruct((BATCH, SEQ, DIM), jnp.float32),
    mesh=plsc.VectorSubcoreMesh(core_axis_name="c", subcore_axis_name="s"),
    scratch_types=[
        pltpu.VMEM((C,), jnp.int32),
        pltpu.VMEM((C,), jnp.int32),
        pltpu.VMEM((2, BPC, SEQ, DIM), jnp.float32),
        pltpu.VMEM((SEQ, DIM), jnp.float32),
        pltpu.SemaphoreType.DMA,
        pltpu.SemaphoreType.DMA,
        pltpu.SemaphoreType.DMA,
        pltpu.SemaphoreType.DMA,
    ],
)
def _sc_lookup(item_hbm, pos_hbm, table_hbm, out_hbm,
               idx0, idx1, buf, pos_v, gsem0, gsem1, ssem0, ssem1):
    cid = lax.axis_index("c")
    sid = lax.axis_index("s")
    wid = sid * NC + cid
    idxs = (idx0, idx1)
    gsem = (gsem0, gsem1)
    ssem = (ssem0, ssem1)

    pltpu.sync_copy(pos_hbm, pos_v)

    def fire_chunk(t, par):
        base = wid * BPW * SEQ + t * C
        pltpu.sync_copy(item_hbm.at[pl.ds(base, C)], idxs[par])
        for b in range(BPC):
            for off, lane_lo, lane_hi in GROUPS:
                v = idxs[par][pl.ds(b * SEQ + off, 16)]
                for i in range(lane_lo, lane_hi):
                    row = v[i]
                    pltpu.async_copy(
                        table_hbm.at[pl.ds(row, 1)],
                        buf.at[par, b, pl.ds(off + i, 1)],
                        gsem[par],
                    )

    def drain_gather(par):
        pltpu.make_async_copy(
            out_hbm.at[pl.ds(0, BPC)], buf.at[par], gsem[par]
        ).wait()

    def drain_store(par):
        pltpu.make_async_copy(
            buf.at[par], out_hbm.at[pl.ds(0, BPC)], ssem[par]
        ).wait()

    fire_chunk(0, 0)

    def step(t, par):
        other = 1 - par

        @pl.when(t >= 1)
        def _():
            drain_store(other)

        @pl.when(t + 1 < CH)
        def _():
            fire_chunk(t + 1, other)

        drain_gather(par)

        def add_s(s, carry):
            pv = [pos_v[s, pl.ds(d * 16, 16)] for d in range(NLANE)]
            for b in range(BPC):
                for d in range(NLANE):
                    sl = pl.ds(d * 16, 16)
                    buf[par, b, s, sl] = buf[par, b, s, sl] + pv[d]
            return carry

        lax.fori_loop(0, SEQ, add_s, 0)

        bb = wid * BPW + t * BPC
        pltpu.async_copy(buf.at[par], out_hbm.at[pl.ds(bb, BPC)], ssem[par])

    def pair(tt, carry):
        step(tt * 2, 0)
        step(tt * 2 + 1, 1)
        return carry

    lax.fori_loop(0, CH // 2, pair, 0)
    drain_store((CH - 1) % 2)


def kernel(item, nbr_mask, i_ids, item_input_lookup, position_embedding):
    idx_flat = item.reshape(-1)
    pos = position_embedding.reshape(SEQ, DIM)
    return _sc_lookup(idx_flat, pos, item_input_lookup)
